# Initial kernel scaffold; baseline (speedup 1.0000x reference)
import jax
import jax.numpy as jnp
from jax.experimental import pallas as pl


def kernel(node_features, relative_positions_sh, senders, receivers, W_shortcut, W_lin_s1, W_lin_s2, W_lin_v):
    raise NotImplementedError("write your pallas kernel here")


# speedup vs baseline: 1.2264x; 1.2264x over previous
import functools

import numpy as np
import jax
import jax.numpy as jnp
from jax import lax
from jax.experimental import pallas as pl
from jax.experimental.pallas import tpu as pltpu

N_NODES = 10000
N_EDGES = 320000
C = 128
TGT_S = 128
TGT_V = 64
S_W = 5 * C
OUT_W = TGT_S + TGT_V * 3


def _p_row(k):
    if k < C:
        return C + k
    kp = k - C
    return 2 * C + C * (kp % 3) + (kp // 3)


def _build_G(W_shortcut, W_lin_s1, W_lin_s2, W_lin_v):
    inv15_16 = 1.0 / (1.5 * 16.0)
    inv_sqrtC = 1.0 / np.sqrt(float(C))
    s_out1 = 1.0 / (1.5 * np.sqrt(float(C)))

    G = jnp.zeros((S_W, OUT_W), dtype=jnp.float32)
    G = G.at[0:C, 0:TGT_S].add(W_lin_s1 * inv15_16 + W_shortcut * inv_sqrtC)
    rows_xb = np.array([_p_row(128 + 256 + j) for j in range(C)], dtype=np.int32)
    G = G.at[rows_xb, 0:TGT_S].add(W_lin_s2 * inv15_16)
    for mo in range(3):
        rows_m = np.array([_p_row(3 * u + mo) for u in range(C)], dtype=np.int32)
        cols_m = np.array([TGT_S + 3 * v + mo for v in range(TGT_V)], dtype=np.int32)
        G = G.at[rows_m[:, None], cols_m[None, :]].add(W_lin_v * s_out1)
    return G


def _final_matmul_kernel(s_ref, cnt_ref, g_ref, out_ref):
    inv = 1.0 / jnp.maximum(cnt_ref[...], 1.0)
    x = s_ref[...] * inv
    out_ref[...] = jnp.dot(x, g_ref[...], preferred_element_type=jnp.float32)


def _final_matmul(S, cnt, G):
    n = S.shape[0]
    blk = 1000
    grid = n // blk
    return pl.pallas_call(
        _final_matmul_kernel,
        grid=(grid,),
        in_specs=[
            pl.BlockSpec((blk, S_W), lambda i: (i, 0)),
            pl.BlockSpec((blk, 1), lambda i: (i, 0)),
            pl.BlockSpec((S_W, OUT_W), lambda i: (0, 0)),
        ],
        out_specs=pl.BlockSpec((blk, OUT_W), lambda i: (i, 0)),
        out_shape=jax.ShapeDtypeStruct((n, OUT_W), jnp.float32),
    )(S, cnt, G)


def kernel(node_features, relative_positions_sh, senders, receivers,
           W_shortcut, W_lin_s1, W_lin_s2, W_lin_v):
    N = node_features.shape[0]
    E = relative_positions_sh.shape[0]

    G = _build_G(W_shortcut, W_lin_s1, W_lin_s2, W_lin_v)

    nb = node_features[senders]
    w = jnp.concatenate([jnp.ones((E, 1), jnp.float32),
                         relative_positions_sh], axis=1)
    msg = (w[:, :, None] * nb[:, None, :]).reshape(E, S_W)
    S = jax.ops.segment_sum(msg, receivers, num_segments=N)
    cnt = jax.ops.segment_sum(jnp.ones((E,), jnp.float32), receivers,
                              num_segments=N)[:, None]

    return _final_matmul(S, cnt, G)


# speedup vs baseline: 1.2723x; 1.0375x over previous
import functools

import numpy as np
import jax
import jax.numpy as jnp
from jax import lax
from jax.experimental import pallas as pl
from jax.experimental.pallas import tpu as pltpu
from jax.experimental.pallas import tpu_sc as plsc

N_NODES = 10000
N_EDGES = 320000
C = 128
TGT_S = 128
TGT_V = 64
S_W = 5 * C
OUT_W = TGT_S + TGT_V * 3

NC = 2
NS = 16
CH = 40
EPT = N_EDGES // (NC * NS)
NCHUNK = EPT // CH
N_PAD = 10240
ROWS_PT = N_PAD // NS
RCH = 40
NRCH = ROWS_PT // RCH


def _p_row(k):
    if k < C:
        return C + k
    kp = k - C
    return 2 * C + C * (kp % 3) + (kp // 3)


def _build_G(W_shortcut, W_lin_s1, W_lin_s2, W_lin_v):
    inv15_16 = 1.0 / (1.5 * 16.0)
    inv_sqrtC = 1.0 / np.sqrt(float(C))
    s_out1 = 1.0 / (1.5 * np.sqrt(float(C)))

    G = jnp.zeros((S_W, OUT_W), dtype=jnp.float32)
    G = G.at[0:C, 0:TGT_S].add(W_lin_s1 * inv15_16 + W_shortcut * inv_sqrtC)
    rows_xb = np.array([_p_row(128 + 256 + j) for j in range(C)], dtype=np.int32)
    G = G.at[rows_xb, 0:TGT_S].add(W_lin_s2 * inv15_16)
    for mo in range(3):
        rows_m = np.array([_p_row(3 * u + mo) for u in range(C)], dtype=np.int32)
        cols_m = np.array([TGT_S + 3 * v + mo for v in range(TGT_V)], dtype=np.int32)
        G = G.at[rows_m[:, None], cols_m[None, :]].add(W_lin_v * s_out1)
    return G.reshape(5, C, OUT_W)


def _sc_body(nf_ref, snd_ref, rcv_ref, w1_ref, w2_ref, w3_ref, w4_ref,
             s_out_ref, cnt_out_ref,
             acc_ref, sidx_ref, ridx_ref, w_ref, rows_ref, ones_ref,
             zbd_ref, zbz_ref, sem):
    cid = lax.axis_index("c")
    sid = lax.axis_index("s")
    tile_base = (cid * NS + sid) * EPT
    r0 = sid * ROWS_PT

    zv = jnp.zeros((16,), jnp.float32)
    ov = jnp.ones((16,), jnp.float32)

    def init_bufs(e, _):
        for j8 in range(8):
            zbz_ref[e, pl.ds(j8 * 16, 16)] = zv
            ones_ref[e, pl.ds(j8 * 16, 16)] = ov
        return 0

    lax.fori_loop(0, RCH, init_bufs, 0)

    for i in range(NRCH):
        pltpu.sync_copy(zbz_ref, acc_ref.at[pl.ds(r0 + i * RCH, RCH), :])
    plsc.subcore_barrier()

    for k in range(6):
        def chunk(j, _):
            base = tile_base + j * CH
            pltpu.sync_copy(rcv_ref.at[pl.ds(base, CH)], ridx_ref)
            if k == 5:
                pltpu.sync_copy(ones_ref, acc_ref.at[ridx_ref], add=True)
                return 0
            pltpu.sync_copy(snd_ref.at[pl.ds(base, CH)], sidx_ref)
            pltpu.async_copy(nf_ref.at[sidx_ref], rows_ref, sem).wait()
            if k == 0:
                pltpu.sync_copy(rows_ref, acc_ref.at[ridx_ref], add=True)
            else:
                wk_ref = (w1_ref, w2_ref, w3_ref, w4_ref)[k - 1]
                pltpu.sync_copy(wk_ref.at[pl.ds(base, CH), :], w_ref)

                def scale(e, _):
                    w = w_ref[e, pl.ds(0, 16)]
                    for j8 in range(8):
                        rows_ref[e, pl.ds(j8 * 16, 16)] = (
                            rows_ref[e, pl.ds(j8 * 16, 16)] * w)
                    return 0

                lax.fori_loop(0, CH, scale, 0)
                pltpu.sync_copy(rows_ref, acc_ref.at[ridx_ref], add=True)
            return 0

        lax.fori_loop(0, NCHUNK, chunk, 0)
        plsc.subcore_barrier()
        for i in range(NRCH):
            roff = r0 + i * RCH
            pltpu.sync_copy(acc_ref.at[pl.ds(roff, RCH), :], zbd_ref)
            if k < 5:
                pltpu.sync_copy(zbz_ref, acc_ref.at[pl.ds(roff, RCH), :])
            if k == 5:
                pltpu.sync_copy(
                    zbd_ref, cnt_out_ref.at[pl.ds(cid * N_PAD + roff, RCH), :])
            else:
                pltpu.sync_copy(
                    zbd_ref, s_out_ref.at[k, pl.ds(cid * N_PAD + roff, RCH), :])
        plsc.subcore_barrier()


def _sc_sums(node_features, senders, receivers, shs):
    mesh = plsc.VectorSubcoreMesh(core_axis_name="c", subcore_axis_name="s",
                                  num_cores=NC, num_subcores=NS)
    f = pl.kernel(
        _sc_body,
        out_type=(
            jax.ShapeDtypeStruct((5, NC * N_PAD, C), jnp.float32),
            jax.ShapeDtypeStruct((NC * N_PAD, C), jnp.float32),
        ),
        mesh=mesh,
        scratch_types=[
            pltpu.VMEM_SHARED((N_PAD, C), jnp.float32),
            pltpu.VMEM((CH,), jnp.int32),
            pltpu.VMEM((CH,), jnp.int32),
            pltpu.VMEM((CH, 16), jnp.float32),
            pltpu.VMEM((CH, C), jnp.float32),
            pltpu.VMEM((CH, C), jnp.float32),
            pltpu.VMEM((RCH, C), jnp.float32),
            pltpu.VMEM((RCH, C), jnp.float32),
            pltpu.SemaphoreType.DMA,
        ],
    )
    return f(node_features, senders, receivers, *shs)


def _final_matmul_kernel(s_ref, cnt_ref, g_ref, out_ref):
    cnt = cnt_ref[0, :, 0] + cnt_ref[1, :, 0]
    inv = (1.0 / jnp.maximum(cnt, 1.0))[:, None]
    acc = jnp.zeros(out_ref.shape, dtype=jnp.float32)
    for k in range(5):
        s = (s_ref[k, 0] + s_ref[k, 1]) * inv
        acc += jnp.dot(s, g_ref[k], preferred_element_type=jnp.float32)
    out_ref[...] = acc


def _final_matmul(S, cnt, G):
    blk = 1024
    grid = N_PAD // blk
    S = S.reshape(5, NC, N_PAD, C)
    cnt = cnt.reshape(NC, N_PAD, C)
    return pl.pallas_call(
        _final_matmul_kernel,
        grid=(grid,),
        in_specs=[
            pl.BlockSpec((5, NC, blk, C), lambda i: (0, 0, i, 0)),
            pl.BlockSpec((NC, blk, C), lambda i: (0, i, 0)),
            pl.BlockSpec((5, C, OUT_W), lambda i: (0, 0, 0)),
        ],
        out_specs=pl.BlockSpec((blk, OUT_W), lambda i: (i, 0)),
        out_shape=jax.ShapeDtypeStruct((N_PAD, OUT_W), jnp.float32),
    )(S, cnt, G)


def kernel(node_features, relative_positions_sh, senders, receivers,
           W_shortcut, W_lin_s1, W_lin_s2, W_lin_v):
    G = _build_G(W_shortcut, W_lin_s1, W_lin_s2, W_lin_v)
    shs = [jnp.broadcast_to(relative_positions_sh[:, i][:, None],
                            (N_EDGES, 16)) for i in range(4)]
    S, cnt = _sc_sums(node_features, senders, receivers, shs)
    return _final_matmul(S, cnt, G)[:N_NODES]


# speedup vs baseline: 2.3453x; 1.8433x over previous
import functools

import numpy as np
import jax
import jax.numpy as jnp
from jax import lax
from jax.experimental import pallas as pl
from jax.experimental.pallas import tpu as pltpu
from jax.experimental.pallas import tpu_sc as plsc

N_NODES = 10000
N_EDGES = 320000
C = 128
TGT_S = 128
TGT_V = 64
S_W = 5 * C
OUT_W = TGT_S + TGT_V * 3

NC = 2
NS = 16
CH = 40
EPT = N_EDGES // (NC * NS)
NCHUNK = EPT // CH
N_PAD = 10240
ROWS_PT = N_PAD // NS
RCH = 40
NRCH = ROWS_PT // RCH


def _p_row(k):
    if k < C:
        return C + k
    kp = k - C
    return 2 * C + C * (kp % 3) + (kp // 3)


def _build_G(W_shortcut, W_lin_s1, W_lin_s2, W_lin_v):
    inv15_16 = 1.0 / (1.5 * 16.0)
    inv_sqrtC = 1.0 / np.sqrt(float(C))
    s_out1 = 1.0 / (1.5 * np.sqrt(float(C)))

    G = jnp.zeros((S_W, OUT_W), dtype=jnp.float32)
    G = G.at[0:C, 0:TGT_S].add(W_lin_s1 * inv15_16 + W_shortcut * inv_sqrtC)
    rows_xb = np.array([_p_row(128 + 256 + j) for j in range(C)], dtype=np.int32)
    G = G.at[rows_xb, 0:TGT_S].add(W_lin_s2 * inv15_16)
    for mo in range(3):
        rows_m = np.array([_p_row(3 * u + mo) for u in range(C)], dtype=np.int32)
        cols_m = np.array([TGT_S + 3 * v + mo for v in range(TGT_V)], dtype=np.int32)
        G = G.at[rows_m[:, None], cols_m[None, :]].add(W_lin_v * s_out1)
    return G.reshape(5, C, OUT_W)


def _sc_body(nf_ref, snd_ref, rcv_ref, w1_ref, w2_ref, w3_ref, w4_ref,
             s_out_ref, cnt_out_ref,
             acc_ref, rowsA, rowsB, scatA, scatB, sidxA, sidxB, wA, wB,
             ridx0, ridx1, ridx2, ridx3,
             semG, semIA, semIB, semWA, semWB, semS,
             semR0, semR1, semR2, semR3):
    cid = lax.axis_index("c")
    sid = lax.axis_index("s")
    tile_ebase = (cid * NS + sid) * EPT
    r0 = sid * ROWS_PT

    rows = (rowsA, rowsB)
    scat = (scatA, scatB)
    sidx = (sidxA, sidxB)
    wbuf = (wA, wB)
    ridx = (ridx0, ridx1, ridx2, ridx3)
    semI = (semIA, semIB)
    semW = (semWA, semWB)
    semR = (semR0, semR1, semR2, semR3)
    wks = (None, w1_ref, w2_ref, w3_ref, w4_ref, None)

    def drain_G():
        pltpu.make_async_copy(nf_ref.at[pl.ds(0, CH), :], rowsA, semG).wait()

    def drain_I(p):
        pltpu.make_async_copy(snd_ref.at[pl.ds(0, CH)], sidx[p], semI[p]).wait()

    def drain_W(p):
        pltpu.make_async_copy(w1_ref.at[pl.ds(0, CH), :], wbuf[p], semW[p]).wait()

    def drain_R(q):
        pltpu.make_async_copy(rcv_ref.at[pl.ds(0, CH)], ridx[q], semR[q]).wait()

    def drain_S():
        pltpu.make_async_copy(nf_ref.at[pl.ds(0, CH), :], scatA, semS).wait()

    def start_sidx(c, p):
        pltpu.async_copy(snd_ref.at[pl.ds(tile_ebase + c * CH, CH)],
                         sidx[p], semI[p])

    def start_ridx(c, q):
        pltpu.async_copy(rcv_ref.at[pl.ds(tile_ebase + c * CH, CH)],
                         ridx[q], semR[q])

    def start_w(k, c, p):
        pltpu.async_copy(wks[k].at[pl.ds(tile_ebase + c * CH, CH), :],
                         wbuf[p], semW[p])

    def start_gather(p):
        pltpu.async_copy(nf_ref.at[sidx[p]], rows[p], semG)

    def start_scatter(src_ref, q):
        pltpu.async_copy(src_ref, acc_ref.at[ridx[q]], semS, add=True)

    def scale(k, p):
        if k == 0:
            def body(e, _):
                for j8 in range(8):
                    scat[p][e, pl.ds(j8 * 16, 16)] = rows[p][e, pl.ds(j8 * 16, 16)]
                return 0
        else:
            def body(e, _):
                w = wbuf[p][e, pl.ds(0, 16)]
                for j8 in range(8):
                    scat[p][e, pl.ds(j8 * 16, 16)] = (
                        rows[p][e, pl.ds(j8 * 16, 16)] * w)
                return 0
        lax.fori_loop(0, CH, body, 0)

    def fill(buf, val):
        v = jnp.full((16,), val, jnp.float32)

        def body(e, _):
            for j8 in range(8):
                buf[e, pl.ds(j8 * 16, 16)] = v
            return 0

        lax.fori_loop(0, CH, body, 0)

    def data_chunk(k, c, p2, p4, drain_s):
        if drain_s:
            drain_S()
        drain_G()
        drain_I(1 - p2)
        start_gather(1 - p2)
        start_sidx(c + 2, p2)
        if k > 0:
            drain_W(p2)
        scale(k, p2)
        if k > 0:
            start_w(k, c + 2, p2)
        drain_R(p4)
        start_scatter(scat[p2], p4)
        start_ridx(c + 2, (p4 + 2) % 4)

    def cnt_chunk(c, p4, drain_s):
        if drain_s:
            drain_S()
        drain_R(p4)
        start_scatter(scatA, p4)
        start_ridx(c + 2, (p4 + 2) % 4)

    fill(scatB, 0.0)
    for i in range(NRCH):
        pltpu.sync_copy(scatB, acc_ref.at[pl.ds(r0 + i * RCH, RCH), :])
    plsc.subcore_barrier()

    for k in range(6):
        zero = jnp.zeros((), jnp.int32)
        if k < 5:
            pltpu.sync_copy(snd_ref.at[pl.ds(tile_ebase, CH)], sidxA)
            start_gather(0)
            start_sidx(zero + 1, 1)
            start_ridx(zero, 0)
            start_ridx(zero + 1, 1)
            if k > 0:
                start_w(k, zero, 0)
                start_w(k, zero + 1, 1)
            data_chunk(k, zero, 0, 0, False)
            data_chunk(k, zero + 1, 1, 1, False)

            def quad(m, _):
                c = 4 * m + 2
                data_chunk(k, c, 0, 2, True)
                data_chunk(k, c + 1, 1, 3, True)
                data_chunk(k, c + 2, 0, 0, True)
                data_chunk(k, c + 3, 1, 1, True)
                return 0

            lax.fori_loop(0, (NCHUNK - 2) // 4, quad, 0)
            drain_S()
            drain_S()
            drain_G()
            drain_I(1)
            drain_R(2)
            drain_R(3)
            if k > 0:
                drain_W(0)
                drain_W(1)
        else:
            fill(scatA, 1.0)
            start_ridx(zero, 0)
            start_ridx(zero + 1, 1)
            cnt_chunk(zero, 0, False)
            cnt_chunk(zero + 1, 1, False)

            def quad(m, _):
                c = 4 * m + 2
                cnt_chunk(c, 2, True)
                cnt_chunk(c + 1, 3, True)
                cnt_chunk(c + 2, 0, True)
                cnt_chunk(c + 3, 1, True)
                return 0

            lax.fori_loop(0, (NCHUNK - 2) // 4, quad, 0)
            drain_S()
            drain_S()
            drain_R(2)
            drain_R(3)
        plsc.subcore_barrier()
        fill(scatB, 0.0)
        for i in range(NRCH):
            roff = r0 + i * RCH
            pltpu.sync_copy(acc_ref.at[pl.ds(roff, RCH), :], rowsA)
            if k < 5:
                pltpu.sync_copy(scatB, acc_ref.at[pl.ds(roff, RCH), :])
            if k == 5:
                pltpu.sync_copy(
                    rowsA, cnt_out_ref.at[pl.ds(cid * N_PAD + roff, RCH), :])
            else:
                pltpu.sync_copy(
                    rowsA, s_out_ref.at[k, pl.ds(cid * N_PAD + roff, RCH), :])
        plsc.subcore_barrier()


def _sc_sums(node_features, senders, receivers, shs):
    mesh = plsc.VectorSubcoreMesh(core_axis_name="c", subcore_axis_name="s",
                                  num_cores=NC, num_subcores=NS)
    f = pl.kernel(
        _sc_body,
        out_type=(
            jax.ShapeDtypeStruct((5, NC * N_PAD, C), jnp.float32),
            jax.ShapeDtypeStruct((NC * N_PAD, C), jnp.float32),
        ),
        mesh=mesh,
        scratch_types=[
            pltpu.VMEM_SHARED((N_PAD, C), jnp.float32),
            pltpu.VMEM((CH, C), jnp.float32),
            pltpu.VMEM((CH, C), jnp.float32),
            pltpu.VMEM((CH, C), jnp.float32),
            pltpu.VMEM((CH, C), jnp.float32),
            pltpu.VMEM((CH,), jnp.int32),
            pltpu.VMEM((CH,), jnp.int32),
            pltpu.VMEM((CH, 16), jnp.float32),
            pltpu.VMEM((CH, 16), jnp.float32),
            pltpu.VMEM((CH,), jnp.int32),
            pltpu.VMEM((CH,), jnp.int32),
            pltpu.VMEM((CH,), jnp.int32),
            pltpu.VMEM((CH,), jnp.int32),
        ] + [pltpu.SemaphoreType.DMA] * 10,
    )
    return f(node_features, senders, receivers, *shs)


def _final_matmul_kernel(s_ref, cnt_ref, g_ref, out_ref):
    cnt = cnt_ref[0, :, 0] + cnt_ref[1, :, 0]
    inv = (1.0 / jnp.maximum(cnt, 1.0))[:, None]
    acc = jnp.zeros(out_ref.shape, dtype=jnp.float32)
    for k in range(5):
        s = (s_ref[k, 0] + s_ref[k, 1]) * inv
        acc += jnp.dot(s, g_ref[k], preferred_element_type=jnp.float32)
    out_ref[...] = acc


def _final_matmul(S, cnt, G):
    blk = 1024
    grid = N_PAD // blk
    S = S.reshape(5, NC, N_PAD, C)
    cnt = cnt.reshape(NC, N_PAD, C)
    return pl.pallas_call(
        _final_matmul_kernel,
        grid=(grid,),
        in_specs=[
            pl.BlockSpec((5, NC, blk, C), lambda i: (0, 0, i, 0)),
            pl.BlockSpec((NC, blk, C), lambda i: (0, i, 0)),
            pl.BlockSpec((5, C, OUT_W), lambda i: (0, 0, 0)),
        ],
        out_specs=pl.BlockSpec((blk, OUT_W), lambda i: (i, 0)),
        out_shape=jax.ShapeDtypeStruct((N_PAD, OUT_W), jnp.float32),
    )(S, cnt, G)


def kernel(node_features, relative_positions_sh, senders, receivers,
           W_shortcut, W_lin_s1, W_lin_s2, W_lin_v):
    G = _build_G(W_shortcut, W_lin_s1, W_lin_s2, W_lin_v)
    pad = 2 * CH
    snd_p = jnp.concatenate([senders, jnp.zeros((pad,), jnp.int32)])
    rcv_p = jnp.concatenate([receivers, jnp.zeros((pad,), jnp.int32)])
    shs = [jnp.concatenate([
        jnp.broadcast_to(relative_positions_sh[:, i][:, None], (N_EDGES, 16)),
        jnp.zeros((pad, 16), jnp.float32)]) for i in range(4)]
    S, cnt = _sc_sums(node_features, snd_p, rcv_p, shs)
    return _final_matmul(S, cnt, G)[:N_NODES]
